# baseline (speedup 1.0000x reference)
import functools

import jax
import jax.numpy as jnp
from jax import lax
from jax.experimental import pallas as pl
from jax.experimental.pallas import tpu as pltpu
from jax.experimental.pallas import tpu_sc as plsc

D = 64
ROW = 2 * D
NC, NS = 2, 16
NW = NC * NS
CHUNK1 = 128
CHUNK2 = 96
NBUF = 4


def _relu(x):
    return jnp.maximum(x, 0.0)


@functools.lru_cache(maxsize=None)
def _build(batch):
    pw = batch // NW
    n1 = (pw * 2) // CHUNK1
    n2 = (pw * 3) // CHUNK2

    mesh = plsc.VectorSubcoreMesh(core_axis_name="c", subcore_axis_name="s")

    @functools.partial(
        pl.kernel,
        mesh=mesh,
        out_type=jax.ShapeDtypeStruct((NW, 16), jnp.float32),
        scratch_types=[
            pltpu.VMEM((n1, CHUNK1), jnp.int32),
            pltpu.VMEM((n2, CHUNK2), jnp.int32),
            pltpu.VMEM((NBUF, CHUNK1, ROW), jnp.float32),
            pltpu.VMEM((16,), jnp.float32),
            pltpu.SemaphoreType.DMA,
            pltpu.SemaphoreType.DMA,
        ],
    )
    def k(nf1_hbm, nf2_hbm, emb_hbm, out_hbm, idx1, idx2, rows, accv,
          isem, sem):
        wid = lax.axis_index("s") * NC + lax.axis_index("c")

        ic1 = pltpu.async_copy(nf1_hbm.at[wid], idx1, isem)
        ic2 = pltpu.async_copy(nf2_hbm.at[wid], idx2, isem)
        ic1.wait()
        ic2.wait()

        def issue(t):
            b = t % NBUF
            if t < n1:
                return pltpu.async_copy(emb_hbm.at[idx1.at[t]],
                                        rows.at[b, pl.ds(0, CHUNK1)], sem)
            g = t - n1
            return pltpu.async_copy(emb_hbm.at[idx2.at[g]],
                                    rows.at[b, pl.ds(0, CHUNK2)], sem)

        def compute_nf1(b, accs):
            def body(m, a):
                out = list(a)
                for j in range(4):
                    cC = rows[b, 2 * m, pl.ds(16 * j, 16)]
                    cO = rows[b, 2 * m, pl.ds(D + 16 * j, 16)]
                    dC = rows[b, 2 * m + 1, pl.ds(16 * j, 16)]
                    dO = rows[b, 2 * m + 1, pl.ds(D + 16 * j, 16)]
                    out[j] = out[j] + (_relu(dC - cC) + _relu(cO - dO)
                                       + _relu(cC - cO) + _relu(dC - dO))
                return tuple(out)

            return lax.fori_loop(0, CHUNK1 // 2, body, accs)

        def compute_nf2(b, accs):
            def body(m, a):
                out = list(a)
                for j in range(4):
                    cC = rows[b, 3 * m, pl.ds(16 * j, 16)]
                    cO = rows[b, 3 * m, pl.ds(D + 16 * j, 16)]
                    dC = rows[b, 3 * m + 1, pl.ds(16 * j, 16)]
                    dO = rows[b, 3 * m + 1, pl.ds(D + 16 * j, 16)]
                    eC = rows[b, 3 * m + 2, pl.ds(16 * j, 16)]
                    eO = rows[b, 3 * m + 2, pl.ds(D + 16 * j, 16)]
                    start_all = jnp.maximum(cC, dC)
                    end_all = jnp.minimum(cO, dO)
                    out[j] = out[j] + (_relu(eC - start_all)
                                       + _relu(end_all - eO)
                                       + _relu(cC - cO) + _relu(dC - dO)
                                       + _relu(eC - eO))
                return tuple(out)

            return lax.fori_loop(0, CHUNK2 // 3, body, accs)

        ntask = n1 + n2
        zero = jnp.zeros((16,), jnp.float32)
        accs = (zero, zero, zero, zero)

        inflight = [issue(t) for t in range(min(NBUF - 1, ntask))]
        for t in range(ntask):
            cp = inflight.pop(0)
            cp.wait()
            nt = t + NBUF - 1
            if nt < ntask:
                inflight.append(issue(nt))
            b = t % NBUF
            if t < n1:
                accs = compute_nf1(b, accs)
            else:
                accs = compute_nf2(b, accs)

        accv[...] = (accs[0] + accs[1]) + (accs[2] + accs[3])
        pltpu.sync_copy(accv, out_hbm.at[wid])

    return k


def kernel(nf1, nf2, classEmb):
    batch = nf1.shape[0]
    pw = batch // NW
    n1 = (pw * 2) // CHUNK1
    n2 = (pw * 3) // CHUNK2
    nf1_r = nf1.reshape(NW, n1, CHUNK1)
    nf2_r = nf2.reshape(NW, n2, CHUNK2)
    out = _build(batch)(nf1_r, nf2_r, classEmb)
    return jnp.sum(out) / jnp.float32(batch)


# speedup vs baseline: 1.5468x; 1.5468x over previous
import functools

import jax
import jax.numpy as jnp
from jax import lax
from jax.experimental import pallas as pl
from jax.experimental.pallas import tpu as pltpu
from jax.experimental.pallas import tpu_sc as plsc

D = 64
ROW = 2 * D
NC, NS = 2, 16
NW = NC * NS
CHUNK = 64
NBUF = 4


def _relu(x):
    return jnp.maximum(x, 0.0)


@functools.lru_cache(maxsize=None)
def _build(batch):
    pw = batch // NW
    nchunk = pw // CHUNK

    mesh = plsc.VectorSubcoreMesh(core_axis_name="c", subcore_axis_name="s")

    @functools.partial(
        pl.kernel,
        mesh=mesh,
        out_type=jax.ShapeDtypeStruct((NW, 16), jnp.float32),
        scratch_types=[
            pltpu.VMEM((2, nchunk, CHUNK), jnp.int32),
            pltpu.VMEM((3, nchunk, CHUNK), jnp.int32),
            pltpu.VMEM((NBUF, CHUNK, ROW), jnp.float32),
            pltpu.VMEM((NBUF, CHUNK, ROW), jnp.float32),
            pltpu.VMEM((NBUF, CHUNK, ROW), jnp.float32),
            pltpu.VMEM((16,), jnp.float32),
            pltpu.SemaphoreType.DMA,
            pltpu.SemaphoreType.DMA,
        ],
    )
    def k(nf1_hbm, nf2_hbm, emb_hbm, out_hbm, idx1, idx2, rc, rd, re, accv,
          isem, sem):
        wid = lax.axis_index("s") * NC + lax.axis_index("c")

        cps = [
            pltpu.async_copy(nf1_hbm.at[c, wid], idx1.at[c], isem)
            for c in range(2)
        ] + [
            pltpu.async_copy(nf2_hbm.at[c, wid], idx2.at[c], isem)
            for c in range(3)
        ]
        for cp in cps:
            cp.wait()

        def issue(t):
            b = t % NBUF
            if t < nchunk:
                return [
                    pltpu.async_copy(emb_hbm.at[idx1.at[0, t]], rc.at[b], sem),
                    pltpu.async_copy(emb_hbm.at[idx1.at[1, t]], rd.at[b], sem),
                ]
            g = t - nchunk
            return [
                pltpu.async_copy(emb_hbm.at[idx2.at[0, g]], rc.at[b], sem),
                pltpu.async_copy(emb_hbm.at[idx2.at[1, g]], rd.at[b], sem),
                pltpu.async_copy(emb_hbm.at[idx2.at[2, g]], re.at[b], sem),
            ]

        def compute_nf1(b, accs):
            def body(i, a):
                out = list(a)
                for j in range(4):
                    cC = rc[b, i, pl.ds(16 * j, 16)]
                    cO = rc[b, i, pl.ds(D + 16 * j, 16)]
                    dC = rd[b, i, pl.ds(16 * j, 16)]
                    dO = rd[b, i, pl.ds(D + 16 * j, 16)]
                    out[j] = out[j] + (_relu(dC - cC) + _relu(cO - dO)
                                       + _relu(cC - cO) + _relu(dC - dO))
                return tuple(out)

            return lax.fori_loop(0, CHUNK, body, accs)

        def compute_nf2(b, accs):
            def body(i, a):
                out = list(a)
                for j in range(4):
                    cC = rc[b, i, pl.ds(16 * j, 16)]
                    cO = rc[b, i, pl.ds(D + 16 * j, 16)]
                    dC = rd[b, i, pl.ds(16 * j, 16)]
                    dO = rd[b, i, pl.ds(D + 16 * j, 16)]
                    eC = re[b, i, pl.ds(16 * j, 16)]
                    eO = re[b, i, pl.ds(D + 16 * j, 16)]
                    start_all = jnp.maximum(cC, dC)
                    end_all = jnp.minimum(cO, dO)
                    out[j] = out[j] + (_relu(eC - start_all)
                                       + _relu(end_all - eO)
                                       + _relu(cC - cO) + _relu(dC - dO)
                                       + _relu(eC - eO))
                return tuple(out)

            return lax.fori_loop(0, CHUNK, body, accs)

        ntask = 2 * nchunk
        zero = jnp.zeros((16,), jnp.float32)
        accs = (zero, zero, zero, zero)

        inflight = [issue(t) for t in range(min(NBUF - 1, ntask))]
        for t in range(ntask):
            for cp in inflight.pop(0):
                cp.wait()
            nt = t + NBUF - 1
            if nt < ntask:
                inflight.append(issue(nt))
            b = t % NBUF
            if t < nchunk:
                accs = compute_nf1(b, accs)
            else:
                accs = compute_nf2(b, accs)

        accv[...] = (accs[0] + accs[1]) + (accs[2] + accs[3])
        pltpu.sync_copy(accv, out_hbm.at[wid])

    return k


def kernel(nf1, nf2, classEmb):
    batch = nf1.shape[0]
    pw = batch // NW
    nchunk = pw // CHUNK
    nf1_t = nf1.T.reshape(2, NW, nchunk, CHUNK)
    nf2_t = nf2.T.reshape(3, NW, nchunk, CHUNK)
    out = _build(batch)(nf1_t, nf2_t, classEmb)
    return jnp.sum(out) / jnp.float32(batch)
